# baseline (speedup 1.0000x reference)
import jax
import jax.numpy as jnp
from jax import lax
from jax.experimental import pallas as pl
from jax.experimental.pallas import tpu as pltpu
from jax.experimental.pallas import tpu_sc as plsc

N = 10000
E = 320000
D = 128

NC = 2
NS = 16
NW = NC * NS
K = 128
EPW = 10240
EP = NW * EPW
NCHUNK = EPW // K
NR = NCHUNK // 2
ACCR = N + 8
DR = 624
TAIL = N - DR * NS
DEGW = 128
Z0 = 128

_MESH = plsc.VectorSubcoreMesh(
    core_axis_name="c", subcore_axis_name="s", num_cores=NC, num_subcores=NS
)


def _zero_rows(ref, nrows, width):
    zv = jnp.zeros((16,), jnp.float32)
    groups = width // 16

    def body(i, _):
        ref[i // groups, pl.ds((i % groups) * 16, 16)] = zv
        return 0

    lax.fori_loop(0, nrows * groups, body, 0)


def _zero_acc(acc_sh, stage_v, s):
    for z in range(4):
        pltpu.sync_copy(stage_v, acc_sh.at[pl.ds(s * DR + z * Z0, Z0)])
    pltpu.sync_copy(
        stage_v.at[pl.ds(0, DR - 4 * Z0)],
        acc_sh.at[pl.ds(s * DR + 4 * Z0, DR - 4 * Z0)],
    )

    @pl.when(s == NS - 1)
    def _():
        pltpu.sync_copy(stage_v.at[pl.ds(0, TAIL)], acc_sh.at[pl.ds(DR * NS, TAIL)])


def _drain_acc(acc_sh, hbm, c, s):
    pltpu.sync_copy(acc_sh.at[pl.ds(s * DR, DR)], hbm.at[c, pl.ds(s * DR, DR)])

    @pl.when(s == NS - 1)
    def _():
        pltpu.sync_copy(
            acc_sh.at[pl.ds(DR * NS, TAIL)], hbm.at[c, pl.ds(DR * NS, TAIL)]
        )


def _deg_body(dst_hbm, degp_hbm, dst_v0, dst_v1, ones_v, stage_v, acc_sh,
              ssem0, ssem1):
    c = lax.axis_index("c")
    s = lax.axis_index("s")
    wid = s * NC + c
    base = wid * EPW

    onev = jnp.full((16,), 1.0, jnp.float32)
    groups = DEGW // 16

    def fill_ones(i, _):
        ones_v[i // groups, pl.ds((i % groups) * 16, 16)] = onev
        return 0

    lax.fori_loop(0, K * groups, fill_ones, 0)
    _zero_rows(stage_v, Z0, DEGW)
    _zero_acc(acc_sh, stage_v, s)
    plsc.subcore_barrier()

    pltpu.sync_copy(dst_hbm.at[pl.ds(base, K)], dst_v0)

    def rnd(r, _):
        sc0 = pltpu.async_copy(ones_v, acc_sh.at[dst_v0], ssem0, add=True)
        pltpu.sync_copy(dst_hbm.at[pl.ds(base + (2 * r + 1) * K, K)], dst_v1)
        sc1 = pltpu.async_copy(ones_v, acc_sh.at[dst_v1], ssem1, add=True)
        sc0.wait()

        @pl.when(r < NR - 1)
        def _():
            pltpu.sync_copy(dst_hbm.at[pl.ds(base + (2 * r + 2) * K, K)], dst_v0)

        sc1.wait()
        return 0

    lax.fori_loop(0, NR, rnd, 0)
    plsc.subcore_barrier()
    _drain_acc(acc_sh, degp_hbm, c, s)


def _agg_body(g_hbm, src_hbm, dst_hbm, accp_hbm,
              src_v0, dst_v0, rows_v0, src_v1, dst_v1, rows_v1, stage_v,
              acc_sh, gsem0, gsem1, ssem0, ssem1):
    c = lax.axis_index("c")
    s = lax.axis_index("s")
    wid = s * NC + c
    base = wid * EPW

    _zero_rows(stage_v, Z0, D)
    _zero_acc(acc_sh, stage_v, s)
    plsc.subcore_barrier()

    def rnd(i, _):
        b = base + i * K
        pltpu.sync_copy(src_hbm.at[pl.ds(b, K)], src_v0)
        pltpu.sync_copy(dst_hbm.at[pl.ds(b, K)], dst_v0)
        pltpu.async_copy(g_hbm.at[src_v0], rows_v0, gsem0).wait()
        pltpu.sync_copy(rows_v0, acc_sh.at[dst_v0], add=True)
        return 0

    lax.fori_loop(0, NCHUNK, rnd, 0)
    plsc.subcore_barrier()
    _drain_acc(acc_sh, accp_hbm, c, s)


_DEG_SCRATCH = [
    pltpu.VMEM((K,), jnp.int32),
    pltpu.VMEM((K,), jnp.int32),
    pltpu.VMEM((K, DEGW), jnp.float32),
    pltpu.VMEM((Z0, DEGW), jnp.float32),
    pltpu.VMEM_SHARED((ACCR, DEGW), jnp.float32),
    pltpu.SemaphoreType.DMA,
    pltpu.SemaphoreType.DMA,
]

_AGG_SCRATCH = [
    pltpu.VMEM((K,), jnp.int32),
    pltpu.VMEM((K,), jnp.int32),
    pltpu.VMEM((K, D), jnp.float32),
    pltpu.VMEM((K,), jnp.int32),
    pltpu.VMEM((K,), jnp.int32),
    pltpu.VMEM((K, D), jnp.float32),
    pltpu.VMEM((Z0, D), jnp.float32),
    pltpu.VMEM_SHARED((ACCR, D), jnp.float32),
    pltpu.SemaphoreType.DMA,
    pltpu.SemaphoreType.DMA,
    pltpu.SemaphoreType.DMA,
    pltpu.SemaphoreType.DMA,
]

_deg_kernel = pl.kernel(
    _deg_body,
    out_type=jax.ShapeDtypeStruct((NC, N, DEGW), jnp.float32),
    mesh=_MESH,
    scratch_types=_DEG_SCRATCH,
)

_agg_kernel = pl.kernel(
    _agg_body,
    out_type=jax.ShapeDtypeStruct((NC, N, D), jnp.float32),
    mesh=_MESH,
    scratch_types=_AGG_SCRATCH,
)


def _tc1_body(feat_ref, w1_ref, degp_ref, g1_ref, dinv_ref):
    deg = degp_ref[0] + degp_ref[1] + 1.0
    dinv16 = lax.rsqrt(deg)
    h = jnp.dot(feat_ref[...], w1_ref[...], preferred_element_type=jnp.float32)
    g1_ref[...] = h * dinv16[:, 0:1]
    dinv_ref[...] = dinv16


def _tc2_body(accp_ref, g1_ref, dinv_ref, b1_ref, w2_ref, g2_ref):
    dinv = dinv_ref[:, 0:1]
    x = (accp_ref[0] + accp_ref[1] + g1_ref[...]) * dinv + b1_ref[...]
    x = jnp.where(x > 0.0, x, jnp.exp(x) - 1.0)
    h2 = jnp.dot(x, w2_ref[...], preferred_element_type=jnp.float32)
    g2_ref[...] = h2 * dinv


def _tc3_body(accp_ref, g2_ref, dinv_ref, b2_ref, out_ref):
    dinv = dinv_ref[:, 0:1]
    out_ref[...] = (accp_ref[0] + accp_ref[1] + g2_ref[...]) * dinv + b2_ref[...]


_tc1 = pl.pallas_call(
    _tc1_body,
    out_shape=[
        jax.ShapeDtypeStruct((N, D), jnp.float32),
        jax.ShapeDtypeStruct((N, DEGW), jnp.float32),
    ],
)

_tc2 = pl.pallas_call(
    _tc2_body,
    out_shape=jax.ShapeDtypeStruct((N, D), jnp.float32),
)

_tc3 = pl.pallas_call(
    _tc3_body,
    out_shape=jax.ShapeDtypeStruct((N, D), jnp.float32),
)


def kernel(feature, edge_index, W1, b1, W2, b2):
    pad = EP - E
    src = jnp.concatenate([edge_index[0], jnp.zeros((pad,), jnp.int32)])
    dst = jnp.concatenate([edge_index[1], jnp.full((pad,), N, jnp.int32)])
    degp = _deg_kernel(dst)
    g1, dinv16 = _tc1(feature, W1, degp)
    acc1 = _agg_kernel(g1, src, dst)
    g2 = _tc2(acc1, g1, dinv16, b1.reshape(1, D), W2)
    acc2 = _agg_kernel(g2, src, dst)
    return _tc3(acc2, g2, dinv16, b2.reshape(1, D))


# speedup vs baseline: 1.1692x; 1.1692x over previous
import jax
import jax.numpy as jnp
from jax import lax
from jax.experimental import pallas as pl
from jax.experimental.pallas import tpu as pltpu
from jax.experimental.pallas import tpu_sc as plsc

N = 10000
E = 320000
D = 128

NC = 2
NS = 16
NW = NC * NS
K = 128
EPW = 10240
EP = NW * EPW
NCHUNK = EPW // K
NR = NCHUNK // 2
NDUMP = 160
ACCR = N + NDUMP
DR = 624
TAIL = N - DR * NS
DEGW = 128
Z0 = 128

_MESH = plsc.VectorSubcoreMesh(
    core_axis_name="c", subcore_axis_name="s", num_cores=NC, num_subcores=NS
)


def _zero_rows(ref, nrows, width):
    zv = jnp.zeros((16,), jnp.float32)
    groups = width // 16

    def body(i, _):
        ref[i // groups, pl.ds((i % groups) * 16, 16)] = zv
        return 0

    lax.fori_loop(0, nrows * groups, body, 0)


def _zero_acc(acc_sh, stage_v, s):
    for z in range(4):
        pltpu.sync_copy(stage_v, acc_sh.at[pl.ds(s * DR + z * Z0, Z0)])
    pltpu.sync_copy(
        stage_v.at[pl.ds(0, DR - 4 * Z0)],
        acc_sh.at[pl.ds(s * DR + 4 * Z0, DR - 4 * Z0)],
    )

    @pl.when(s == NS - 1)
    def _():
        pltpu.sync_copy(stage_v.at[pl.ds(0, TAIL)], acc_sh.at[pl.ds(DR * NS, TAIL)])


def _drain_acc(acc_sh, hbm, c, s):
    pltpu.sync_copy(acc_sh.at[pl.ds(s * DR, DR)], hbm.at[c, pl.ds(s * DR, DR)])

    @pl.when(s == NS - 1)
    def _():
        pltpu.sync_copy(
            acc_sh.at[pl.ds(DR * NS, TAIL)], hbm.at[c, pl.ds(DR * NS, TAIL)]
        )


def _deg_body(dst_hbm, degp_hbm, dst_v0, dst_v1, ones_v, stage_v, acc_sh,
              ssem0, ssem1):
    c = lax.axis_index("c")
    s = lax.axis_index("s")
    wid = s * NC + c
    base = wid * EPW

    onev = jnp.full((16,), 1.0, jnp.float32)
    groups = DEGW // 16

    def fill_ones(i, _):
        ones_v[i // groups, pl.ds((i % groups) * 16, 16)] = onev
        return 0

    lax.fori_loop(0, K * groups, fill_ones, 0)
    _zero_rows(stage_v, Z0, DEGW)
    _zero_acc(acc_sh, stage_v, s)
    plsc.subcore_barrier()

    pltpu.sync_copy(dst_hbm.at[pl.ds(base, K)], dst_v0)

    def rnd(r, _):
        sc0 = pltpu.async_copy(ones_v, acc_sh.at[dst_v0], ssem0, add=True)
        pltpu.sync_copy(dst_hbm.at[pl.ds(base + (2 * r + 1) * K, K)], dst_v1)
        sc1 = pltpu.async_copy(ones_v, acc_sh.at[dst_v1], ssem1, add=True)
        sc0.wait()

        @pl.when(r < NR - 1)
        def _():
            pltpu.sync_copy(dst_hbm.at[pl.ds(base + (2 * r + 2) * K, K)], dst_v0)

        sc1.wait()
        return 0

    lax.fori_loop(0, NR, rnd, 0)
    plsc.subcore_barrier()
    _drain_acc(acc_sh, degp_hbm, c, s)


def _agg_body(g_hbm, src_hbm, dst_hbm, accp_hbm,
              src_v0, dst_v0, rows_v0, src_v1, dst_v1, rows_v1,
              acc_sh, gsem0, gsem1, ssem0, ssem1):
    c = lax.axis_index("c")
    s = lax.axis_index("s")
    wid = s * NC + c
    base = wid * EPW

    _zero_rows(rows_v0, Z0, D)
    _zero_acc(acc_sh, rows_v0, s)
    plsc.subcore_barrier()

    pltpu.sync_copy(src_hbm.at[pl.ds(base, K)], src_v0)
    pltpu.sync_copy(dst_hbm.at[pl.ds(base, K)], dst_v0)
    pltpu.async_copy(g_hbm.at[src_v0], rows_v0, gsem0)
    pltpu.sync_copy(src_hbm.at[pl.ds(base + K, K)], src_v1)
    pltpu.sync_copy(dst_hbm.at[pl.ds(base + K, K)], dst_v1)
    pltpu.async_copy(g_hbm.at[src_v1], rows_v1, gsem1)

    def rnd(r, _):
        pltpu.make_async_copy(g_hbm.at[src_v0], rows_v0, gsem0).wait()
        sc0 = pltpu.async_copy(rows_v0, acc_sh.at[dst_v0], ssem0, add=True)
        pltpu.make_async_copy(g_hbm.at[src_v1], rows_v1, gsem1).wait()
        sc1 = pltpu.async_copy(rows_v1, acc_sh.at[dst_v1], ssem1, add=True)

        sc0.wait()

        @pl.when(r < NR - 1)
        def _():
            pltpu.sync_copy(src_hbm.at[pl.ds(base + (2 * r + 2) * K, K)], src_v0)
            pltpu.sync_copy(dst_hbm.at[pl.ds(base + (2 * r + 2) * K, K)], dst_v0)
            pltpu.async_copy(g_hbm.at[src_v0], rows_v0, gsem0)

        sc1.wait()

        @pl.when(r < NR - 1)
        def _():
            pltpu.sync_copy(src_hbm.at[pl.ds(base + (2 * r + 3) * K, K)], src_v1)
            pltpu.sync_copy(dst_hbm.at[pl.ds(base + (2 * r + 3) * K, K)], dst_v1)
            pltpu.async_copy(g_hbm.at[src_v1], rows_v1, gsem1)

        return 0

    lax.fori_loop(0, NR, rnd, 0)
    plsc.subcore_barrier()
    _drain_acc(acc_sh, accp_hbm, c, s)


_DEG_SCRATCH = [
    pltpu.VMEM((K,), jnp.int32),
    pltpu.VMEM((K,), jnp.int32),
    pltpu.VMEM((K, DEGW), jnp.float32),
    pltpu.VMEM((Z0, DEGW), jnp.float32),
    pltpu.VMEM_SHARED((ACCR, DEGW), jnp.float32),
    pltpu.SemaphoreType.DMA,
    pltpu.SemaphoreType.DMA,
]

_AGG_SCRATCH = [
    pltpu.VMEM((K,), jnp.int32),
    pltpu.VMEM((K,), jnp.int32),
    pltpu.VMEM((K, D), jnp.float32),
    pltpu.VMEM((K,), jnp.int32),
    pltpu.VMEM((K,), jnp.int32),
    pltpu.VMEM((K, D), jnp.float32),
    pltpu.VMEM_SHARED((ACCR, D), jnp.float32),
    pltpu.SemaphoreType.DMA,
    pltpu.SemaphoreType.DMA,
    pltpu.SemaphoreType.DMA,
    pltpu.SemaphoreType.DMA,
]

_deg_kernel = pl.kernel(
    _deg_body,
    out_type=jax.ShapeDtypeStruct((NC, N, DEGW), jnp.float32),
    mesh=_MESH,
    scratch_types=_DEG_SCRATCH,
)

_agg_kernel = pl.kernel(
    _agg_body,
    out_type=jax.ShapeDtypeStruct((NC, N, D), jnp.float32),
    mesh=_MESH,
    scratch_types=_AGG_SCRATCH,
)


def _tc1_body(feat_ref, w1_ref, degp_ref, g1_ref, dinv_ref):
    deg = degp_ref[0] + degp_ref[1] + 1.0
    dinv16 = lax.rsqrt(deg)
    h = jnp.dot(feat_ref[...], w1_ref[...], preferred_element_type=jnp.float32)
    g1_ref[...] = h * dinv16[:, 0:1]
    dinv_ref[...] = dinv16


def _tc2_body(accp_ref, g1_ref, dinv_ref, b1_ref, w2_ref, g2_ref):
    dinv = dinv_ref[:, 0:1]
    x = (accp_ref[0] + accp_ref[1] + g1_ref[...]) * dinv + b1_ref[...]
    x = jnp.where(x > 0.0, x, jnp.exp(x) - 1.0)
    h2 = jnp.dot(x, w2_ref[...], preferred_element_type=jnp.float32)
    g2_ref[...] = h2 * dinv


def _tc3_body(accp_ref, g2_ref, dinv_ref, b2_ref, out_ref):
    dinv = dinv_ref[:, 0:1]
    out_ref[...] = (accp_ref[0] + accp_ref[1] + g2_ref[...]) * dinv + b2_ref[...]


_tc1 = pl.pallas_call(
    _tc1_body,
    out_shape=[
        jax.ShapeDtypeStruct((N, D), jnp.float32),
        jax.ShapeDtypeStruct((N, DEGW), jnp.float32),
    ],
)

_tc2 = pl.pallas_call(
    _tc2_body,
    out_shape=jax.ShapeDtypeStruct((N, D), jnp.float32),
)

_tc3 = pl.pallas_call(
    _tc3_body,
    out_shape=jax.ShapeDtypeStruct((N, D), jnp.float32),
)


def kernel(feature, edge_index, W1, b1, W2, b2):
    pad = EP - E
    src = jnp.concatenate([edge_index[0], jnp.zeros((pad,), jnp.int32)])
    dump = N + (jnp.arange(pad, dtype=jnp.int32) % NDUMP)
    dst = jnp.concatenate([edge_index[1], dump])
    degp = _deg_kernel(dst)
    g1, dinv16 = _tc1(feature, W1, degp)
    acc1 = _agg_kernel(g1, src, dst)
    g2 = _tc2(acc1, g1, dinv16, b1.reshape(1, D), W2)
    acc2 = _agg_kernel(g2, src, dst)
    return _tc3(acc2, g2, dinv16, b2.reshape(1, D))


# speedup vs baseline: 2.5369x; 2.1698x over previous
import jax
import jax.numpy as jnp
from jax import lax
from jax.experimental import pallas as pl
from jax.experimental.pallas import tpu as pltpu
from jax.experimental.pallas import tpu_sc as plsc

N = 10000
E = 320000
D = 128

NC = 2
NS = 16
NW = NC * NS
K = 80
EPW = E // NW
NCHUNK = EPW // K
NR = (NCHUNK - 1) // 2
ACCR = N
DR = 624
TAIL = N - DR * NS
DEGW = 128
Z0 = 128

_MESH = plsc.VectorSubcoreMesh(
    core_axis_name="c", subcore_axis_name="s", num_cores=NC, num_subcores=NS
)


def _zero_rows(ref, nrows, width):
    zv = jnp.zeros((16,), jnp.float32)
    groups = width // 16

    def body(i, _):
        ref[i // groups, pl.ds((i % groups) * 16, 16)] = zv
        return 0

    lax.fori_loop(0, nrows * groups, body, 0)


def _zero_acc(acc_sh, stage_v, s):
    for z in range(4):
        pltpu.sync_copy(stage_v, acc_sh.at[pl.ds(s * DR + z * Z0, Z0)])
    pltpu.sync_copy(
        stage_v.at[pl.ds(0, DR - 4 * Z0)],
        acc_sh.at[pl.ds(s * DR + 4 * Z0, DR - 4 * Z0)],
    )

    @pl.when(s == NS - 1)
    def _():
        pltpu.sync_copy(stage_v.at[pl.ds(0, TAIL)], acc_sh.at[pl.ds(DR * NS, TAIL)])


def _drain_acc(acc_sh, hbm, c, s):
    pltpu.sync_copy(acc_sh.at[pl.ds(s * DR, DR)], hbm.at[c, pl.ds(s * DR, DR)])

    @pl.when(s == NS - 1)
    def _():
        pltpu.sync_copy(
            acc_sh.at[pl.ds(DR * NS, TAIL)], hbm.at[c, pl.ds(DR * NS, TAIL)]
        )


def _deg_body(dst_hbm, degp_hbm, dst_v0, dst_v1, ones_v, stage_v, acc_sh,
              ssem0, ssem1):
    c = lax.axis_index("c")
    s = lax.axis_index("s")
    wid = s * NC + c
    base = wid * EPW

    onev = jnp.full((16,), 1.0, jnp.float32)
    groups = DEGW // 16

    def fill_ones(i, _):
        ones_v[i // groups, pl.ds((i % groups) * 16, 16)] = onev
        return 0

    lax.fori_loop(0, K * groups, fill_ones, 0)
    _zero_rows(stage_v, Z0, DEGW)
    _zero_acc(acc_sh, stage_v, s)
    plsc.subcore_barrier()

    pltpu.sync_copy(dst_hbm.at[pl.ds(base, K)], dst_v0)

    def rnd(r, _):
        sc0 = pltpu.async_copy(ones_v, acc_sh.at[dst_v0], ssem0, add=True)
        pltpu.sync_copy(dst_hbm.at[pl.ds(base + (2 * r + 1) * K, K)], dst_v1)
        sc1 = pltpu.async_copy(ones_v, acc_sh.at[dst_v1], ssem1, add=True)
        sc0.wait()
        pltpu.sync_copy(dst_hbm.at[pl.ds(base + (2 * r + 2) * K, K)], dst_v0)
        sc1.wait()
        return 0

    lax.fori_loop(0, NR, rnd, 0)
    pltpu.sync_copy(ones_v, acc_sh.at[dst_v0], add=True)
    plsc.subcore_barrier()
    _drain_acc(acc_sh, degp_hbm, c, s)


def _agg_body(g_hbm, src_hbm, dst_hbm, accp_hbm,
              src_v0, dst_v0, rows_v0, src_v1, dst_v1, rows_v1, stage_v,
              acc_sh, gsem0, gsem1, ssem0, ssem1):
    c = lax.axis_index("c")
    s = lax.axis_index("s")
    wid = s * NC + c
    base = wid * EPW

    _zero_rows(stage_v, Z0, D)
    _zero_acc(acc_sh, stage_v, s)
    plsc.subcore_barrier()

    pltpu.sync_copy(src_hbm.at[pl.ds(base, K)], src_v0)
    pltpu.sync_copy(dst_hbm.at[pl.ds(base, K)], dst_v0)
    pltpu.async_copy(g_hbm.at[src_v0], rows_v0, gsem0)
    pltpu.sync_copy(src_hbm.at[pl.ds(base + K, K)], src_v1)
    pltpu.sync_copy(dst_hbm.at[pl.ds(base + K, K)], dst_v1)
    pltpu.async_copy(g_hbm.at[src_v1], rows_v1, gsem1)

    def rnd(r, _):
        pltpu.make_async_copy(g_hbm.at[src_v0], rows_v0, gsem0).wait()
        sc0 = pltpu.async_copy(rows_v0, acc_sh.at[dst_v0], ssem0, add=True)
        pltpu.make_async_copy(g_hbm.at[src_v1], rows_v1, gsem1).wait()
        sc1 = pltpu.async_copy(rows_v1, acc_sh.at[dst_v1], ssem1, add=True)

        sc0.wait()
        pltpu.sync_copy(src_hbm.at[pl.ds(base + (2 * r + 2) * K, K)], src_v0)
        pltpu.sync_copy(dst_hbm.at[pl.ds(base + (2 * r + 2) * K, K)], dst_v0)
        pltpu.async_copy(g_hbm.at[src_v0], rows_v0, gsem0)

        sc1.wait()

        @pl.when(r < NR - 1)
        def _():
            pltpu.sync_copy(src_hbm.at[pl.ds(base + (2 * r + 3) * K, K)], src_v1)
            pltpu.sync_copy(dst_hbm.at[pl.ds(base + (2 * r + 3) * K, K)], dst_v1)
            pltpu.async_copy(g_hbm.at[src_v1], rows_v1, gsem1)

        return 0

    lax.fori_loop(0, NR, rnd, 0)
    pltpu.make_async_copy(g_hbm.at[src_v0], rows_v0, gsem0).wait()
    pltpu.sync_copy(rows_v0, acc_sh.at[dst_v0], add=True)
    plsc.subcore_barrier()
    _drain_acc(acc_sh, accp_hbm, c, s)


_DEG_SCRATCH = [
    pltpu.VMEM((K,), jnp.int32),
    pltpu.VMEM((K,), jnp.int32),
    pltpu.VMEM((K, DEGW), jnp.float32),
    pltpu.VMEM((Z0, DEGW), jnp.float32),
    pltpu.VMEM_SHARED((ACCR, DEGW), jnp.float32),
    pltpu.SemaphoreType.DMA,
    pltpu.SemaphoreType.DMA,
]

_AGG_SCRATCH = [
    pltpu.VMEM((K,), jnp.int32),
    pltpu.VMEM((K,), jnp.int32),
    pltpu.VMEM((K, D), jnp.float32),
    pltpu.VMEM((K,), jnp.int32),
    pltpu.VMEM((K,), jnp.int32),
    pltpu.VMEM((K, D), jnp.float32),
    pltpu.VMEM((Z0, D), jnp.float32),
    pltpu.VMEM_SHARED((ACCR, D), jnp.float32),
    pltpu.SemaphoreType.DMA,
    pltpu.SemaphoreType.DMA,
    pltpu.SemaphoreType.DMA,
    pltpu.SemaphoreType.DMA,
]

_deg_kernel = pl.kernel(
    _deg_body,
    out_type=jax.ShapeDtypeStruct((NC, N, DEGW), jnp.float32),
    mesh=_MESH,
    scratch_types=_DEG_SCRATCH,
)

_agg_kernel = pl.kernel(
    _agg_body,
    out_type=jax.ShapeDtypeStruct((NC, N, D), jnp.float32),
    mesh=_MESH,
    scratch_types=_AGG_SCRATCH,
)


def _tc1_body(feat_ref, w1_ref, degp_ref, g1_ref, dinv_ref):
    deg = degp_ref[0] + degp_ref[1] + 1.0
    dinv16 = lax.rsqrt(deg)
    h = jnp.dot(feat_ref[...], w1_ref[...], preferred_element_type=jnp.float32)
    g1_ref[...] = h * dinv16[:, 0:1]
    dinv_ref[...] = dinv16


def _tc2_body(accp_ref, g1_ref, dinv_ref, b1_ref, w2_ref, g2_ref):
    dinv = dinv_ref[:, 0:1]
    x = (accp_ref[0] + accp_ref[1] + g1_ref[...]) * dinv + b1_ref[...]
    x = jnp.where(x > 0.0, x, jnp.exp(x) - 1.0)
    h2 = jnp.dot(x, w2_ref[...], preferred_element_type=jnp.float32)
    g2_ref[...] = h2 * dinv


def _tc3_body(accp_ref, g2_ref, dinv_ref, b2_ref, out_ref):
    dinv = dinv_ref[:, 0:1]
    out_ref[...] = (accp_ref[0] + accp_ref[1] + g2_ref[...]) * dinv + b2_ref[...]


_tc1 = pl.pallas_call(
    _tc1_body,
    out_shape=[
        jax.ShapeDtypeStruct((N, D), jnp.float32),
        jax.ShapeDtypeStruct((N, DEGW), jnp.float32),
    ],
)

_tc2 = pl.pallas_call(
    _tc2_body,
    out_shape=jax.ShapeDtypeStruct((N, D), jnp.float32),
)

_tc3 = pl.pallas_call(
    _tc3_body,
    out_shape=jax.ShapeDtypeStruct((N, D), jnp.float32),
)


def kernel(feature, edge_index, W1, b1, W2, b2):
    src = edge_index[0]
    dst = edge_index[1]
    degp = _deg_kernel(dst)
    g1, dinv16 = _tc1(feature, W1, degp)
    acc1 = _agg_kernel(g1, src, dst)
    g2 = _tc2(acc1, g1, dinv16, b1.reshape(1, D), W2)
    acc2 = _agg_kernel(g2, src, dst)
    return _tc3(acc2, g2, dinv16, b2.reshape(1, D))


# speedup vs baseline: 3.0660x; 1.2085x over previous
import jax
import jax.numpy as jnp
from jax import lax
from jax.experimental import pallas as pl
from jax.experimental.pallas import tpu as pltpu
from jax.experimental.pallas import tpu_sc as plsc

N = 10000
E = 320000
D = 128

NC = 2
NS = 16
NW = NC * NS
K = 80
EPW = E // NW
NCHUNK = EPW // K
NBUF = 4
NR = (NCHUNK - 1) // NBUF
ACCR = N
DR = 624
TAIL = N - DR * NS
DEGW = 128
Z0 = 128

_MESH = plsc.VectorSubcoreMesh(
    core_axis_name="c", subcore_axis_name="s", num_cores=NC, num_subcores=NS
)


def _zero_rows(ref, nrows, width):
    zv = jnp.zeros((16,), jnp.float32)
    groups = width // 16

    def body(i, _):
        ref[i // groups, pl.ds((i % groups) * 16, 16)] = zv
        return 0

    lax.fori_loop(0, nrows * groups, body, 0)


def _zero_acc(acc_sh, stage_v, s):
    for z in range(4):
        pltpu.sync_copy(stage_v, acc_sh.at[pl.ds(s * DR + z * Z0, Z0)])
    pltpu.sync_copy(
        stage_v.at[pl.ds(0, DR - 4 * Z0)],
        acc_sh.at[pl.ds(s * DR + 4 * Z0, DR - 4 * Z0)],
    )

    @pl.when(s == NS - 1)
    def _():
        pltpu.sync_copy(stage_v.at[pl.ds(0, TAIL)], acc_sh.at[pl.ds(DR * NS, TAIL)])


def _zero_acc80(acc_sh, stage80, s):
    for z in range(7):
        pltpu.sync_copy(stage80, acc_sh.at[pl.ds(s * DR + z * 80, 80)])
    pltpu.sync_copy(stage80.at[pl.ds(0, 64)], acc_sh.at[pl.ds(s * DR + 560, 64)])

    @pl.when(s == NS - 1)
    def _():
        pltpu.sync_copy(stage80.at[pl.ds(0, TAIL)], acc_sh.at[pl.ds(DR * NS, TAIL)])


def _drain_acc(acc_sh, hbm, c, s):
    pltpu.sync_copy(acc_sh.at[pl.ds(s * DR, DR)], hbm.at[c, pl.ds(s * DR, DR)])

    @pl.when(s == NS - 1)
    def _():
        pltpu.sync_copy(
            acc_sh.at[pl.ds(DR * NS, TAIL)], hbm.at[c, pl.ds(DR * NS, TAIL)]
        )


def _deg_body(dst_hbm, degp_hbm, dst_v0, dst_v1, ones_v, stage_v, acc_sh,
              ssem0, ssem1):
    c = lax.axis_index("c")
    s = lax.axis_index("s")
    wid = s * NC + c
    base = wid * EPW

    onev = jnp.full((16,), 1.0, jnp.float32)
    groups = DEGW // 16

    def fill_ones(i, _):
        ones_v[i // groups, pl.ds((i % groups) * 16, 16)] = onev
        return 0

    lax.fori_loop(0, K * groups, fill_ones, 0)
    _zero_rows(stage_v, Z0, DEGW)
    _zero_acc(acc_sh, stage_v, s)
    plsc.subcore_barrier()

    pltpu.sync_copy(dst_hbm.at[pl.ds(base, K)], dst_v0)

    def rnd(r, _):
        sc0 = pltpu.async_copy(ones_v, acc_sh.at[dst_v0], ssem0, add=True)
        pltpu.sync_copy(dst_hbm.at[pl.ds(base + (2 * r + 1) * K, K)], dst_v1)
        sc1 = pltpu.async_copy(ones_v, acc_sh.at[dst_v1], ssem1, add=True)
        sc0.wait()
        pltpu.sync_copy(dst_hbm.at[pl.ds(base + (2 * r + 2) * K, K)], dst_v0)
        sc1.wait()
        return 0

    lax.fori_loop(0, NR, rnd, 0)
    pltpu.sync_copy(ones_v, acc_sh.at[dst_v0], add=True)
    plsc.subcore_barrier()
    _drain_acc(acc_sh, degp_hbm, c, s)


def _agg_body(g_hbm, src_hbm, dst_hbm, accp_hbm,
              src_vs, dst_vs, rows_vs, acc_sh, gsems, ssems):
    c = lax.axis_index("c")
    s = lax.axis_index("s")
    wid = s * NC + c
    base = wid * EPW

    _zero_rows(rows_vs[0], K, D)
    _zero_acc80(acc_sh, rows_vs[0], s)
    plsc.subcore_barrier()

    for b in range(NBUF):
        pltpu.sync_copy(src_hbm.at[pl.ds(base + b * K, K)], src_vs[b])
        pltpu.sync_copy(dst_hbm.at[pl.ds(base + b * K, K)], dst_vs[b])
        pltpu.async_copy(g_hbm.at[src_vs[b]], rows_vs[b], gsems[b])

    def rnd(r, _):
        for b in range(NBUF):
            pltpu.make_async_copy(g_hbm.at[src_vs[b]], rows_vs[b], gsems[b]).wait()
            pltpu.async_copy(rows_vs[b], acc_sh.at[dst_vs[b]], ssems[b], add=True)
        for b in range(NBUF):
            pltpu.make_async_copy(rows_vs[b], acc_sh.at[dst_vs[b]], ssems[b]).wait()
            nxt = base + (NBUF * r + NBUF + b) * K

            def refill():
                pltpu.sync_copy(src_hbm.at[pl.ds(nxt, K)], src_vs[b])
                pltpu.sync_copy(dst_hbm.at[pl.ds(nxt, K)], dst_vs[b])
                pltpu.async_copy(g_hbm.at[src_vs[b]], rows_vs[b], gsems[b])

            if b == 0:
                refill()
            else:
                pl.when(r < NR - 1)(refill)
        return 0

    lax.fori_loop(0, NR, rnd, 0)
    pltpu.make_async_copy(g_hbm.at[src_vs[0]], rows_vs[0], gsems[0]).wait()
    pltpu.sync_copy(rows_vs[0], acc_sh.at[dst_vs[0]], add=True)
    plsc.subcore_barrier()
    _drain_acc(acc_sh, accp_hbm, c, s)


_DEG_SCRATCH = [
    pltpu.VMEM((K,), jnp.int32),
    pltpu.VMEM((K,), jnp.int32),
    pltpu.VMEM((K, DEGW), jnp.float32),
    pltpu.VMEM((Z0, DEGW), jnp.float32),
    pltpu.VMEM_SHARED((ACCR, DEGW), jnp.float32),
    pltpu.SemaphoreType.DMA,
    pltpu.SemaphoreType.DMA,
]

_AGG_SCRATCH = [
    [pltpu.VMEM((K,), jnp.int32) for _ in range(NBUF)],
    [pltpu.VMEM((K,), jnp.int32) for _ in range(NBUF)],
    [pltpu.VMEM((K, D), jnp.float32) for _ in range(NBUF)],
    pltpu.VMEM_SHARED((ACCR, D), jnp.float32),
    [pltpu.SemaphoreType.DMA for _ in range(NBUF)],
    [pltpu.SemaphoreType.DMA for _ in range(NBUF)],
]

_deg_kernel = pl.kernel(
    _deg_body,
    out_type=jax.ShapeDtypeStruct((NC, N, DEGW), jnp.float32),
    mesh=_MESH,
    scratch_types=_DEG_SCRATCH,
)

_agg_kernel = pl.kernel(
    _agg_body,
    out_type=jax.ShapeDtypeStruct((NC, N, D), jnp.float32),
    mesh=_MESH,
    scratch_types=_AGG_SCRATCH,
)


def _tc1_body(feat_ref, w1_ref, degp_ref, g1_ref, dinv_ref):
    deg = degp_ref[0] + degp_ref[1] + 1.0
    dinv16 = lax.rsqrt(deg)
    h = jnp.dot(feat_ref[...], w1_ref[...], preferred_element_type=jnp.float32)
    g1_ref[...] = h * dinv16[:, 0:1]
    dinv_ref[...] = dinv16


def _tc2_body(accp_ref, g1_ref, dinv_ref, b1_ref, w2_ref, g2_ref):
    dinv = dinv_ref[:, 0:1]
    x = (accp_ref[0] + accp_ref[1] + g1_ref[...]) * dinv + b1_ref[...]
    x = jnp.where(x > 0.0, x, jnp.exp(x) - 1.0)
    h2 = jnp.dot(x, w2_ref[...], preferred_element_type=jnp.float32)
    g2_ref[...] = h2 * dinv


def _tc3_body(accp_ref, g2_ref, dinv_ref, b2_ref, out_ref):
    dinv = dinv_ref[:, 0:1]
    out_ref[...] = (accp_ref[0] + accp_ref[1] + g2_ref[...]) * dinv + b2_ref[...]


_tc1 = pl.pallas_call(
    _tc1_body,
    out_shape=[
        jax.ShapeDtypeStruct((N, D), jnp.float32),
        jax.ShapeDtypeStruct((N, DEGW), jnp.float32),
    ],
)

_tc2 = pl.pallas_call(
    _tc2_body,
    out_shape=jax.ShapeDtypeStruct((N, D), jnp.float32),
)

_tc3 = pl.pallas_call(
    _tc3_body,
    out_shape=jax.ShapeDtypeStruct((N, D), jnp.float32),
)


def kernel(feature, edge_index, W1, b1, W2, b2):
    src = edge_index[0]
    dst = edge_index[1]
    degp = _deg_kernel(dst)
    g1, dinv16 = _tc1(feature, W1, degp)
    acc1 = _agg_kernel(g1, src, dst)
    g2 = _tc2(acc1, g1, dinv16, b1.reshape(1, D), W2)
    acc2 = _agg_kernel(g2, src, dst)
    return _tc3(acc2, g2, dinv16, b2.reshape(1, D))
